# Initial kernel scaffold; baseline (speedup 1.0000x reference)
import functools

import jax
import jax.numpy as jnp
from jax import lax
from jax.experimental import pallas as pl
from jax.experimental.pallas import tpu as pltpu
from jax.experimental.pallas import tpu_sc as plsc

B = 4096
S = 200
D = 64
V = 1000
SCALE = 8.0

NC = 2
NS = 16
NW = NC * NS
B_PER_W = B // NW

_mesh = plsc.VectorSubcoreMesh(core_axis_name="c", subcore_axis_name="s")


@functools.partial(
    pl.kernel,
    out_type=jax.ShapeDtypeStruct((B, S, D), jnp.float32),
    mesh=_mesh,
    scratch_types=[
        pltpu.VMEM((V * D,), jnp.float32),
        pltpu.VMEM((S, D), jnp.float32),
        pltpu.VMEM((B_PER_W, D), jnp.float32),
        pltpu.VMEM((16, S), jnp.int32),
        pltpu.VMEM((B_PER_W,), jnp.int32),
        pltpu.VMEM((B_PER_W,), jnp.float32),
        pltpu.VMEM((D,), jnp.float32),
        pltpu.VMEM((D,), jnp.float32),
        pltpu.VMEM((S, D), jnp.float32),
        pltpu.VMEM((S, D), jnp.float32),
        pltpu.SemaphoreType.DMA,
        pltpu.SemaphoreType.DMA,
        pltpu.SemaphoreType.DMA,
    ],
)
def _sc_embed(smiles_hbm, ads_hbm, chemo_hbm, table_hbm, ads_table_hbm,
              pos_hbm, w_hbm, cb_hbm, out_hbm,
              t_v, p_v, r_v, idx_v, adsid_v, chemo_v, w_v, cb_v,
              out0_v, out1_v, sem_g, sem_o0, sem_o1):
    wid = lax.axis_index("s") * NC + lax.axis_index("c")
    base = wid * B_PER_W

    pltpu.sync_copy(table_hbm, t_v)
    pltpu.sync_copy(pos_hbm, p_v)
    pltpu.sync_copy(w_hbm, w_v)
    pltpu.sync_copy(cb_hbm, cb_v)
    pltpu.sync_copy(ads_hbm.at[pl.ds(base, B_PER_W)], adsid_v)
    pltpu.sync_copy(chemo_hbm.at[pl.ds(base, B_PER_W)], chemo_v)
    pltpu.async_copy(ads_table_hbm.at[adsid_v], r_v, sem_g).wait()

    @pl.loop(0, B_PER_W)
    def _r_loop(b):
        chv = plsc.load_gather(chemo_v, [jnp.full((16,), b, jnp.int32)])
        chv8 = chv * SCALE
        for j in range(4):
            sl = pl.ds(16 * j, 16)
            a = r_v[b, sl]
            r_v[b, sl] = a * SCALE + chv8 * w_v[sl] + cb_v[sl] * SCALE

    col = [lax.iota(jnp.int32, 16) + 16 * j for j in range(4)]
    zero = jnp.zeros((16,), jnp.float32)

    out_bufs = (out0_v, out1_v)
    out_sems = (sem_o0, sem_o1)

    @pl.loop(0, 8)
    def _chunk(c):
        pltpu.sync_copy(smiles_hbm.at[pl.ds(base + c * 16, 16)], idx_v)

        @pl.loop(0, 8)
        def _pair(pr):
            for slot in range(2):
                r = pr * 2 + slot
                b = c * 16 + r
                obuf = out_bufs[slot]
                osem = out_sems[slot]

                @pl.when(jnp.logical_or(c > 0, pr > 0))
                def _():
                    pltpu.make_async_copy(obuf, out_hbm.at[0], osem).wait()

                rrow = [r_v[b, pl.ds(16 * j, 16)] for j in range(4)]
                rv = jnp.full((16,), r, jnp.int32)

                @pl.loop(0, 25)
                def _toks(k):
                    s0 = k * 8
                    for u in range(8):
                        s = s0 + u
                        tokv = plsc.load_gather(
                            idx_v, [rv, jnp.full((16,), s, jnp.int32)])
                        m = tokv != 0
                        tok64 = lax.shift_left(tokv, 6)
                        for j in range(4):
                            g = plsc.load_gather(t_v, [tok64 + col[j]])
                            pj = p_v[s, pl.ds(16 * j, 16)]
                            val = jnp.where(m, g * SCALE + (pj + rrow[j]), zero)
                            obuf[s, pl.ds(16 * j, 16)] = val

                pltpu.async_copy(obuf, out_hbm.at[base + b], osem)

    pltpu.make_async_copy(out0_v, out_hbm.at[0], sem_o0).wait()
    pltpu.make_async_copy(out1_v, out_hbm.at[0], sem_o1).wait()


def kernel(smiles, adsorbent, chemometrics, smile_table, ads_table, pos_table,
           chemo_W, chemo_b):
    return _sc_embed(smiles, adsorbent, chemometrics,
                     smile_table.reshape(V * D), ads_table, pos_table,
                     chemo_W.reshape(D), chemo_b)


# speedup vs baseline: 2.1394x; 2.1394x over previous
import functools

import jax
import jax.numpy as jnp
from jax import lax
from jax.experimental import pallas as pl
from jax.experimental.pallas import tpu as pltpu
from jax.experimental.pallas import tpu_sc as plsc

B = 4096
S = 200
D = 64
V = 1000
SCALE = 8.0

NC = 2
NS = 16
NW = NC * NS
B_PER_W = B // NW

_mesh = plsc.VectorSubcoreMesh(core_axis_name="c", subcore_axis_name="s")


@functools.partial(
    pl.kernel,
    out_type=jax.ShapeDtypeStruct((B * S * D,), jnp.float32),
    mesh=_mesh,
    compiler_params=pltpu.CompilerParams(needs_layout_passes=False),
    scratch_types=[
        pltpu.VMEM((V * D,), jnp.float32),
        pltpu.VMEM((S * D,), jnp.float32),
        pltpu.VMEM((B_PER_W * D,), jnp.float32),
        pltpu.VMEM((16 * S,), jnp.int32),
        pltpu.VMEM((B_PER_W,), jnp.int32),
        pltpu.VMEM((B_PER_W,), jnp.float32),
        pltpu.VMEM((D,), jnp.float32),
        pltpu.VMEM((D,), jnp.float32),
        pltpu.VMEM((S * D,), jnp.float32),
        pltpu.VMEM((S * D,), jnp.float32),
        pltpu.SemaphoreType.DMA,
        pltpu.SemaphoreType.DMA,
    ],
)
def _sc_embed(smiles_hbm, ads_hbm, chemo_hbm, table_hbm, ads_table_hbm,
              pos_hbm, w_hbm, cb_hbm, out_hbm,
              t_v, p_v, r_v, idx_v, adsid_v, chemo_v, w_v, cb_v,
              out0_v, out1_v, sem_o0, sem_o1):
    wid = lax.axis_index("s") * NC + lax.axis_index("c")
    base = wid * B_PER_W

    col = [lax.iota(jnp.int32, 16) + 16 * j for j in range(4)]
    zero = jnp.zeros((16,), jnp.float32)

    pltpu.sync_copy(ads_table_hbm, t_v)
    pltpu.sync_copy(pos_hbm, p_v)
    pltpu.sync_copy(w_hbm, w_v)
    pltpu.sync_copy(cb_hbm, cb_v)
    pltpu.sync_copy(ads_hbm.at[pl.ds(base, B_PER_W)], adsid_v)
    pltpu.sync_copy(chemo_hbm.at[pl.ds(base, B_PER_W)], chemo_v)

    @pl.loop(0, B_PER_W)
    def _r_loop(b):
        bv = jnp.full((16,), b, jnp.int32)
        chv8 = plsc.load_gather(chemo_v, [bv]) * SCALE
        aid64 = lax.shift_left(plsc.load_gather(adsid_v, [bv]), 6)
        for j in range(4):
            a = plsc.load_gather(t_v, [aid64 + col[j]])
            sl = pl.ds(16 * j, 16)
            r_v[pl.ds(b * D + 16 * j, 16)] = (
                a * SCALE + chv8 * w_v[sl] + cb_v[sl] * SCALE)

    pltpu.sync_copy(table_hbm, t_v)

    out_bufs = (out0_v, out1_v)
    out_sems = (sem_o0, sem_o1)

    @pl.loop(0, 8)
    def _chunk(c):
        pltpu.sync_copy(smiles_hbm.at[pl.ds((base + c * 16) * S, 16 * S)],
                        idx_v)

        @pl.loop(0, 8)
        def _pair(pr):
            for slot in range(2):
                r = pr * 2 + slot
                b = c * 16 + r
                obuf = out_bufs[slot]
                osem = out_sems[slot]

                @pl.when(jnp.logical_or(c > 0, pr > 0))
                def _():
                    pltpu.make_async_copy(
                        obuf, out_hbm.at[pl.ds(0, S * D)], osem).wait()

                rrow = [r_v[pl.ds(b * D + 16 * j, 16)] for j in range(4)]
                roff = r * S

                @pl.loop(0, 25)
                def _toks(k):
                    s0 = k * 8
                    for u in range(8):
                        s = s0 + u
                        tokv = plsc.load_gather(
                            idx_v, [jnp.full((16,), roff + s, jnp.int32)])
                        m = tokv != 0
                        tok64 = lax.shift_left(tokv, 6)
                        for j in range(4):
                            g = plsc.load_gather(t_v, [tok64 + col[j]])
                            pj = p_v[pl.ds(s * D + 16 * j, 16)]
                            val = jnp.where(m, g * SCALE + (pj + rrow[j]), zero)
                            obuf[pl.ds(s * D + 16 * j, 16)] = val

                pltpu.async_copy(
                    obuf, out_hbm.at[pl.ds((base + b) * S * D, S * D)], osem)

    pltpu.make_async_copy(out0_v, out_hbm.at[pl.ds(0, S * D)], sem_o0).wait()
    pltpu.make_async_copy(out1_v, out_hbm.at[pl.ds(0, S * D)], sem_o1).wait()


def kernel(smiles, adsorbent, chemometrics, smile_table, ads_table, pos_table,
           chemo_W, chemo_b):
    flat = _sc_embed(smiles.reshape(B * S), adsorbent, chemometrics,
                     smile_table.reshape(V * D), ads_table.reshape(V * D),
                     pos_table.reshape(S * D), chemo_W.reshape(D), chemo_b)
    return flat.reshape(B, S, D)


# speedup vs baseline: 4.3653x; 2.0405x over previous
import functools

import jax
import jax.numpy as jnp
from jax import lax
from jax.experimental import pallas as pl
from jax.experimental.pallas import tpu as pltpu
from jax.experimental.pallas import tpu_sc as plsc

B = 4096
S = 200
D = 64
V = 1000
SCALE = 8.0

NC = 2
NS = 16
NW = NC * NS
B_PER_W = B // NW

_mesh = plsc.VectorSubcoreMesh(core_axis_name="c", subcore_axis_name="s")


@functools.partial(
    pl.kernel,
    out_type=jax.ShapeDtypeStruct((B, S * D), jnp.float32),
    mesh=_mesh,
    compiler_params=pltpu.CompilerParams(needs_layout_passes=False),
    scratch_types=[
        pltpu.VMEM((V * D,), jnp.float32),
        pltpu.VMEM((S * D,), jnp.float32),
        pltpu.VMEM((B_PER_W * D,), jnp.float32),
        pltpu.VMEM((16 * S,), jnp.int32),
        pltpu.VMEM((B_PER_W,), jnp.int32),
        pltpu.VMEM((B_PER_W,), jnp.float32),
        pltpu.VMEM((D,), jnp.float32),
        pltpu.VMEM((D,), jnp.float32),
        pltpu.VMEM((S * D,), jnp.float32),
        pltpu.VMEM((S * D,), jnp.float32),
        pltpu.SemaphoreType.DMA,
        pltpu.SemaphoreType.DMA,
    ],
)
def _sc_embed(smiles_hbm, ads_hbm, chemo_hbm, table_hbm, ads_table_hbm,
              pos_hbm, w_hbm, cb_hbm, out_hbm,
              t_v, p_v, r_v, idx_v, adsid_v, chemo_v, w_v, cb_v,
              out0_v, out1_v, sem_o0, sem_o1):
    wid = lax.axis_index("s") * NC + lax.axis_index("c")
    base = wid * B_PER_W

    col = [lax.iota(jnp.int32, 16) + 16 * j for j in range(4)]
    lane = [jnp.full((16,), i, jnp.int32) for i in range(16)]
    zero = jnp.zeros((16,), jnp.float32)

    pltpu.sync_copy(ads_table_hbm, t_v)
    pltpu.sync_copy(pos_hbm, p_v)
    pltpu.sync_copy(w_hbm, w_v)
    pltpu.sync_copy(cb_hbm, cb_v)
    pltpu.sync_copy(ads_hbm.at[pl.ds(base, B_PER_W)], adsid_v)
    pltpu.sync_copy(chemo_hbm.at[pl.ds(base, B_PER_W)], chemo_v)

    @plsc.parallel_loop(0, B_PER_W)
    def _r_loop(b):
        bv = jnp.full((16,), b, jnp.int32)
        chv8 = plsc.load_gather(chemo_v, [bv]) * SCALE
        aid64 = lax.shift_left(plsc.load_gather(adsid_v, [bv]), 6)
        for j in range(4):
            a = plsc.load_gather(t_v, [aid64 + col[j]])
            sl = pl.ds(16 * j, 16)
            r_v[pl.ds(b * D + 16 * j, 16)] = (
                a * SCALE + chv8 * w_v[sl] + cb_v[sl] * SCALE)

    pltpu.sync_copy(table_hbm, t_v)

    @plsc.parallel_loop(0, V * D, step=128)
    def _scale(i):
        for j in range(8):
            sl = pl.ds(i + 16 * j, 16)
            t_v[sl] = t_v[sl] * SCALE

    out_bufs = (out0_v, out1_v)
    out_sems = (sem_o0, sem_o1)

    @pl.loop(0, 8)
    def _chunk(c):
        pltpu.sync_copy(smiles_hbm.at[pl.ds((base + c * 16) * S, 16 * S)],
                        idx_v)

        @pl.loop(0, 8)
        def _pair(pr):
            for slot in range(2):
                r = pr * 2 + slot
                b = c * 16 + r
                obuf = out_bufs[slot]
                osem = out_sems[slot]

                @pl.when(jnp.logical_or(c > 0, pr > 0))
                def _():
                    pltpu.make_async_copy(
                        obuf, out_hbm.at[0], osem).wait()

                rrow = [r_v[pl.ds(b * D + 16 * j, 16)] for j in range(4)]
                robase = r * S

                def emit_group(s0, i0, n):
                    for i in range(i0, i0 + n):
                        tokb = plsc.load_gather(
                            idx_v, [jnp.full((16,), robase + s0 + i,
                                             jnp.int32)])
                        t64 = lax.shift_left(tokb, 6)
                        m = t64 != 0
                        po = (s0 + i) * D
                        for j in range(4):
                            g = plsc.load_gather(t_v, [t64 + col[j]])
                            pj = p_v[pl.ds(po + 16 * j, 16)]
                            val = jnp.where(m, g + (pj + rrow[j]), zero)
                            obuf[pl.ds(po + 16 * j, 16)] = val

                @plsc.parallel_loop(0, 192, step=16)
                def _toks(s0):
                    emit_group(s0, 0, 16)

                emit_group(184, 8, 8)

                pltpu.async_copy(obuf, out_hbm.at[base + b], osem)

    pltpu.make_async_copy(out0_v, out_hbm.at[0], sem_o0).wait()
    pltpu.make_async_copy(out1_v, out_hbm.at[0], sem_o1).wait()


def kernel(smiles, adsorbent, chemometrics, smile_table, ads_table, pos_table,
           chemo_W, chemo_b):
    flat = _sc_embed(smiles.reshape(B * S), adsorbent, chemometrics,
                     smile_table.reshape(V * D), ads_table.reshape(V * D),
                     pos_table.reshape(S * D), chemo_W.reshape(D), chemo_b)
    return flat.reshape(B, S, D)


# speedup vs baseline: 7.5327x; 1.7256x over previous
import functools

import jax
import jax.numpy as jnp
from jax import lax
from jax.experimental import pallas as pl
from jax.experimental.pallas import tpu as pltpu
from jax.experimental.pallas import tpu_sc as plsc

B = 4096
S = 200
D = 64
V = 1000
SCALE = 8.0

NC = 2
NS = 16
NW = NC * NS
B_PER_W = B // NW
RCHUNK = 4

_mesh = plsc.VectorSubcoreMesh(core_axis_name="c", subcore_axis_name="s")


@functools.partial(
    pl.kernel,
    out_type=jax.ShapeDtypeStruct((B, S * D), jnp.float32),
    mesh=_mesh,
    compiler_params=pltpu.CompilerParams(needs_layout_passes=False),
    scratch_types=[
        pltpu.VMEM((V * D,), jnp.float32),
        pltpu.VMEM((S * D,), jnp.float32),
        pltpu.VMEM((B_PER_W * D,), jnp.float32),
        pltpu.VMEM((RCHUNK * S + 16,), jnp.int32),
        pltpu.VMEM((B_PER_W,), jnp.int32),
        pltpu.VMEM((B_PER_W,), jnp.float32),
        pltpu.VMEM((D,), jnp.float32),
        pltpu.VMEM((D,), jnp.float32),
        pltpu.VMEM((S * D,), jnp.float32),
        pltpu.VMEM((S * D,), jnp.float32),
        pltpu.SemaphoreType.DMA,
        pltpu.SemaphoreType.DMA,
    ],
)
def _sc_embed(smiles_hbm, ads_hbm, chemo_hbm, table_hbm, ads_table_hbm,
              pos_hbm, w_hbm, cb_hbm, out_hbm,
              t_v, p_v, r_v, idx_s, adsid_v, chemo_v, w_v, cb_v,
              out0_v, out1_v, sem_o0, sem_o1):
    wid = lax.axis_index("s") * NC + lax.axis_index("c")
    base = wid * B_PER_W

    iota = lax.iota(jnp.int32, 16)
    col = [iota + 16 * j for j in range(4)]
    zero = jnp.zeros((16,), jnp.float32)

    pltpu.sync_copy(ads_table_hbm, t_v)
    pltpu.sync_copy(pos_hbm, p_v)
    pltpu.sync_copy(w_hbm, w_v)
    pltpu.sync_copy(cb_hbm, cb_v)
    pltpu.sync_copy(ads_hbm.at[pl.ds(base, B_PER_W)], adsid_v)
    pltpu.sync_copy(chemo_hbm.at[pl.ds(base, B_PER_W)], chemo_v)

    @plsc.parallel_loop(0, B_PER_W)
    def _r_loop(b):
        bv = jnp.full((16,), b, jnp.int32)
        chv8 = plsc.load_gather(chemo_v, [bv]) * SCALE
        aid64 = lax.shift_left(plsc.load_gather(adsid_v, [bv]), 6)
        for j in range(4):
            a = plsc.load_gather(t_v, [aid64 + col[j]])
            sl = pl.ds(16 * j, 16)
            r_v[pl.ds(b * D + 16 * j, 16)] = (
                a * SCALE + chv8 * w_v[sl] + cb_v[sl] * SCALE)

    pltpu.sync_copy(table_hbm, t_v)

    @plsc.parallel_loop(0, V * D, step=128)
    def _scale(i):
        for j in range(8):
            sl = pl.ds(i + 16 * j, 16)
            t_v[sl] = t_v[sl] * SCALE

    out_bufs = (out0_v, out1_v)
    out_sems = (sem_o0, sem_o1)

    @pl.loop(0, B_PER_W // RCHUNK)
    def _chunk(c):
        pltpu.sync_copy(
            smiles_hbm.at[pl.ds((base + c * RCHUNK) * S, RCHUNK * S)],
            idx_s.at[pl.ds(0, RCHUNK * S)])

        @pl.loop(0, RCHUNK // 2)
        def _pair(pr):
            for slot in range(2):
                r = pr * 2 + slot
                b = c * RCHUNK + r
                obuf = out_bufs[slot]
                osem = out_sems[slot]

                @pl.when(jnp.logical_or(c > 0, pr > 0))
                def _():
                    pltpu.make_async_copy(
                        obuf, out_hbm.at[0], osem).wait()

                rrow = [r_v[pl.ds(b * D + 16 * j, 16)] for j in range(4)]
                robase = r * S

                @plsc.parallel_loop(0, S, step=8)
                def _toks(s0):
                    t16 = idx_s[pl.ds(robase + s0, 16)]
                    for u in range(8):
                        s = s0 + u
                        tok64 = t16[u] * D
                        t64v = jnp.full((16,), tok64, jnp.int32)
                        m = t64v != 0
                        tidx = t64v + iota
                        po = s * D
                        for j in range(4):
                            g = plsc.load_gather(
                                t_v.at[pl.ds(16 * j, V * D - 16 * j)], [tidx])
                            pj = p_v[pl.ds(po + 16 * j, 16)]
                            val = jnp.where(m, g + (pj + rrow[j]), zero)
                            obuf[pl.ds(po + 16 * j, 16)] = val

                pltpu.async_copy(obuf, out_hbm.at[base + b], osem)

    pltpu.make_async_copy(out0_v, out_hbm.at[0], sem_o0).wait()
    pltpu.make_async_copy(out1_v, out_hbm.at[0], sem_o1).wait()


def kernel(smiles, adsorbent, chemometrics, smile_table, ads_table, pos_table,
           chemo_W, chemo_b):
    flat = _sc_embed(smiles.reshape(B * S), adsorbent, chemometrics,
                     smile_table.reshape(V * D), ads_table.reshape(V * D),
                     pos_table.reshape(S * D), chemo_W.reshape(D), chemo_b)
    return flat.reshape(B, S, D)
